# baseline (speedup 1.0000x reference)
import functools
import jax
import jax.numpy as jnp
from jax import lax
from jax.experimental import pallas as pl
from jax.experimental.pallas import tpu as pltpu
from jax.experimental.pallas import tpu_sc as plsc

H = 12
DH = 64
L = 2048
D = 768
K = 16
G = 8
B = 2
NJ = D // 128
NT = NJ // 2
DP = NT * 128
LP = 2176
TLA = 128
TL = 128
SCALE = DH ** -0.5

NW = 32
CH = 4
IDX_CH = CH * K
NBUF = 2


def _pack_bf16_pair(c0, c1):
    w0 = lax.bitcast_convert_type(c0, jnp.uint32)
    w1 = lax.bitcast_convert_type(c1, jnp.uint32)
    r0 = (w0 + 0x7FFF + ((w0 >> 16) & 1)) >> 16
    r1 = (w1 + 0x7FFF + ((w1 >> 16) & 1)) >> 16
    return r0 | (r1 << 16)


def _unpack_bf16_pair(w):
    c0 = lax.bitcast_convert_type(w << 16, jnp.float32)
    c1 = lax.bitcast_convert_type(w & jnp.uint32(0xFFFF0000), jnp.float32)
    return c0, c1


def _proj_body(x_ref, wq_ref, wk_ref, wv_ref, q_ref, k_ref, v_ref):
    x = x_ref[0]
    q_ref[0] = jnp.dot(x, wq_ref[...], preferred_element_type=jnp.float32)
    kk = jnp.dot(x, wk_ref[...], preferred_element_type=jnp.float32)
    vv = jnp.dot(x, wv_ref[...], preferred_element_type=jnp.float32)
    for t in range(NT):
        lo = slice(2 * t * 128, (2 * t + 1) * 128)
        hi = slice((2 * t + 1) * 128, (2 * t + 2) * 128)
        dst = slice(t * 128, (t + 1) * 128)
        k_ref[0, :, dst] = _pack_bf16_pair(kk[:, lo], kk[:, hi])
        v_ref[0, :, dst] = _pack_bf16_pair(vv[:, lo], vv[:, hi])


def _sc_gather_body(lseg, kp_hbm, vp_hbm, idx_hbm, kg_hbm, vg_hbm,
                    idx_v, kbuf, vbuf, ks0, ks1, vs0, vs1):
    nchunk = B * lseg // CH // NW
    ksem = (ks0, ks1)
    vsem = (vs0, vs1)
    wid = lax.axis_index("s") * 2 + lax.axis_index("c")
    row0 = wid * (B * lseg // NW)
    boff = (row0 // lseg) * LP
    chunk0 = wid * nchunk

    pltpu.sync_copy(idx_hbm.at[pl.ds(chunk0, nchunk)], idx_v)

    def adjust(c, carry):
        for h in range(IDX_CH // 16):
            sl = pl.ds(h * 16, 16)
            idx_v[c, sl] = idx_v[c, sl] + boff
        return carry

    lax.fori_loop(0, nchunk, adjust, 0)

    def fire(c, s):
        pltpu.async_copy(kp_hbm.at[idx_v.at[c]], kbuf.at[s], ksem[s])
        pltpu.async_copy(vp_hbm.at[idx_v.at[c]], vbuf.at[s], vsem[s])

    def drain(c, s):
        pltpu.make_async_copy(kp_hbm.at[idx_v.at[c]], kbuf.at[s],
                              ksem[s]).wait()
        pltpu.make_async_copy(vp_hbm.at[idx_v.at[c]], vbuf.at[s],
                              vsem[s]).wait()
        out_off = (chunk0 + c) * IDX_CH
        pltpu.sync_copy(kbuf.at[s], kg_hbm.at[pl.ds(out_off, IDX_CH)])
        pltpu.sync_copy(vbuf.at[s], vg_hbm.at[pl.ds(out_off, IDX_CH)])

    for s in range(NBUF):
        fire(s, s)

    def group(gi, carry):
        c = gi * NBUF
        for s in range(NBUF):
            drain(c + s, s)
            fire(c + NBUF + s, s)
        return carry

    lax.fori_loop(0, (nchunk - NBUF) // NBUF, group, 0)
    for s in range(NBUF):
        drain(nchunk - NBUF + s, s)


def _sj(j):
    d_over = (lax.broadcasted_iota(jnp.int32, (128, H), 0) + j * 128) // DH
    h_ids = lax.broadcasted_iota(jnp.int32, (128, H), 1)
    return (d_over == h_ids).astype(jnp.float32)


def _stj(j):
    d_over = (lax.broadcasted_iota(jnp.int32, (H, 128), 1) + j * 128) // DH
    h_ids = lax.broadcasted_iota(jnp.int32, (H, 128), 0)
    return (d_over == h_ids).astype(jnp.float32)


def _unpack_cols(ref, n):
    del n
    chunks = []
    for t in range(NT):
        w = ref[0, :, t * 128:(t + 1) * 128]
        c0, c1 = _unpack_bf16_pair(w)
        chunks.extend([c0, c1])
    return chunks


def _attn_body(q_ref, kg_ref, vg_ref, kself_ref, vself_ref, gk_ref, gv_ref,
               dist_ref, par_ref, wout_ref, bout_ref, o_ref):
    q = q_ref[0] * SCALE

    kg_c = _unpack_cols(kg_ref, TL * K)
    ks_c = _unpack_cols(kself_ref, TL)
    gk_c = _unpack_cols(gk_ref, G)

    s_n = jnp.zeros((TL * K, H), jnp.float32)
    s_s = jnp.zeros((TL, H), jnp.float32)
    s_g = jnp.zeros((TL * G, H), jnp.float32)
    for j in range(NJ):
        Sj = _sj(j)
        qj = q[:, j * 128:(j + 1) * 128]
        qnj = jnp.broadcast_to(qj[:, None, :],
                               (TL, K, 128)).reshape(TL * K, 128)
        qgj = jnp.broadcast_to(qj[:, None, :],
                               (TL, G, 128)).reshape(TL * G, 128)
        ggj = jnp.broadcast_to(gk_c[j][None, :, :],
                               (TL, G, 128)).reshape(TL * G, 128)
        s_n += jnp.dot(qnj * kg_c[j], Sj, preferred_element_type=jnp.float32)
        s_s += jnp.dot(qj * ks_c[j], Sj, preferred_element_type=jnp.float32)
        s_g += jnp.dot(qgj * ggj, Sj, preferred_element_type=jnp.float32)

    inv2s = par_ref[0:1, 0:H].reshape(1, 1, H)
    gbias = par_ref[1:2, 0:1].reshape(1, 1, 1)
    dist = dist_ref[0]
    s_n3 = s_n.reshape(TL, K, H) + (dist * dist)[:, :, None] * inv2s
    s_g3 = s_g.reshape(TL, G, H) + gbias

    m = jnp.maximum(jnp.maximum(s_s, s_n3.max(axis=1)), s_g3.max(axis=1))
    e_s = jnp.exp(s_s - m)
    e_n = jnp.exp(s_n3 - m[:, None, :])
    e_g = jnp.exp(s_g3 - m[:, None, :])
    inv_den = 1.0 / (e_s + e_n.sum(axis=1) + e_g.sum(axis=1))
    p_s = e_s * inv_den
    p_n = (e_n * inv_den[:, None, :]).reshape(TL * K, H)
    p_g = (e_g * inv_den[:, None, :]).reshape(TL * G, H)

    vg_c = _unpack_cols(vg_ref, TL * K)
    vs_c = _unpack_cols(vself_ref, TL)
    gv_c = _unpack_cols(gv_ref, G)

    outs = []
    for j in range(NJ):
        Stj = _stj(j)
        gvje = jnp.broadcast_to(gv_c[j][None, :, :],
                                (TL, G, 128)).reshape(TL * G, 128)
        oj = jnp.dot(p_s, Stj, preferred_element_type=jnp.float32) * vs_c[j]
        oj += (jnp.dot(p_n, Stj, preferred_element_type=jnp.float32) * vg_c[j]
               ).reshape(TL, K, 128).sum(axis=1)
        oj += (jnp.dot(p_g, Stj, preferred_element_type=jnp.float32) * gvje
               ).reshape(TL, G, 128).sum(axis=1)
        outs.append(oj)
    out = jnp.concatenate(outs, axis=1)

    o_ref[0] = (jnp.dot(out, wout_ref[...], preferred_element_type=jnp.float32)
                + bout_ref[0:1, :])


@functools.cache
def _make_sc_gather(lseg):
    return pl.kernel(
        functools.partial(_sc_gather_body, lseg),
        mesh=plsc.VectorSubcoreMesh(core_axis_name="c", subcore_axis_name="s"),
        out_type=[jax.ShapeDtypeStruct((B * lseg * K, DP), jnp.uint32)] * 2,
        scratch_types=[
            pltpu.VMEM((B * lseg // CH // NW, IDX_CH), jnp.int32),
            pltpu.VMEM((NBUF, IDX_CH, DP), jnp.uint32),
            pltpu.VMEM((NBUF, IDX_CH, DP), jnp.uint32),
            pltpu.SemaphoreType.DMA,
            pltpu.SemaphoreType.DMA,
            pltpu.SemaphoreType.DMA,
            pltpu.SemaphoreType.DMA,
        ],
    )


def kernel(spatial, topk_indices, rpe, self_rpe, distances, global_latents,
           Wq, Wk, Wv, Wout, b_out, log_sigma, global_bias):
    xall = jnp.concatenate(
        [spatial, global_latents,
         jnp.zeros((B, LP - L - G, D), spatial.dtype)], axis=1)

    q_all, kp, vp = pl.pallas_call(
        _proj_body,
        grid=(B, LP // TLA),
        in_specs=[
            pl.BlockSpec((1, TLA, D), lambda b, i: (b, i, 0)),
            pl.BlockSpec((D, D), lambda b, i: (0, 0)),
            pl.BlockSpec((D, D), lambda b, i: (0, 0)),
            pl.BlockSpec((D, D), lambda b, i: (0, 0)),
        ],
        out_specs=[
            pl.BlockSpec((1, TLA, D), lambda b, i: (b, i, 0)),
            pl.BlockSpec((1, TLA, DP), lambda b, i: (b, i, 0)),
            pl.BlockSpec((1, TLA, DP), lambda b, i: (b, i, 0)),
        ],
        out_shape=[
            jax.ShapeDtypeStruct((B, LP, D), jnp.float32),
            jax.ShapeDtypeStruct((B, LP, DP), jnp.uint32),
            jax.ShapeDtypeStruct((B, LP, DP), jnp.uint32),
        ],
    )(xall, Wq, Wk, Wv)

    params = jnp.zeros((8, 128), jnp.float32)
    params = params.at[0, :H].set(-0.5 * jnp.exp(-2.0 * log_sigma))
    params = params.at[1, 0].set(global_bias)
    bout8 = jnp.broadcast_to(b_out[None, :], (8, D))
    gk = kp[:, L:L + G]
    gv = vp[:, L:L + G]
    kp_flat = kp.reshape(B * LP, DP)
    vp_flat = vp.reshape(B * LP, DP)
    idx32 = topk_indices.astype(jnp.int32)

    idx_chunks = idx32.reshape(B * L // CH, IDX_CH)
    kg_flat, vg_flat = _make_sc_gather(L)(kp_flat, vp_flat, idx_chunks)
    kg = kg_flat.reshape(B, L * K, DP)
    vg = vg_flat.reshape(B, L * K, DP)

    out = pl.pallas_call(
        _attn_body,
        grid=(B, L // TL),
        in_specs=[
            pl.BlockSpec((1, TL, D), lambda b, i: (b, i, 0)),
            pl.BlockSpec((1, TL * K, DP), lambda b, i: (b, i, 0)),
            pl.BlockSpec((1, TL * K, DP), lambda b, i: (b, i, 0)),
            pl.BlockSpec((1, TL, DP), lambda b, i: (b, i, 0)),
            pl.BlockSpec((1, TL, DP), lambda b, i: (b, i, 0)),
            pl.BlockSpec((1, G, DP), lambda b, i: (b, 0, 0)),
            pl.BlockSpec((1, G, DP), lambda b, i: (b, 0, 0)),
            pl.BlockSpec((1, TL, K), lambda b, i: (b, i, 0)),
            pl.BlockSpec((8, 128), lambda b, i: (0, 0)),
            pl.BlockSpec((D, D), lambda b, i: (0, 0)),
            pl.BlockSpec((8, D), lambda b, i: (0, 0)),
        ],
        out_specs=pl.BlockSpec((1, TL, D), lambda b, i: (b, i, 0)),
        out_shape=jax.ShapeDtypeStruct((B, L, D), jnp.float32),
    )(q_all[:, :L], kg, vg, kp[:, :L], vp[:, :L], gk, gv,
      distances, params, Wout, bout8)
    return out


# speedup vs baseline: 1.0680x; 1.0680x over previous
import functools
import functools as ft
import jax
import jax.numpy as jnp
from jax import lax
from jax.experimental import pallas as pl
from jax.experimental.pallas import tpu as pltpu
from jax.experimental.pallas import tpu_sc as plsc

H = 12
DH = 64
L = 2048
D = 768
K = 16
G = 8
B = 2
NJ = D // 128
NT = NJ // 2
DP = NT * 128
LP = 2176
TLA = 128
TL = 128
SCALE = DH ** -0.5

NW = 32
CH = 4
IDX_CH = CH * K
NBUF = 2


def _pack_bf16_pair(c0, c1):
    w0 = lax.bitcast_convert_type(c0, jnp.uint32)
    w1 = lax.bitcast_convert_type(c1, jnp.uint32)
    r0 = (w0 + 0x7FFF + ((w0 >> 16) & 1)) >> 16
    r1 = (w1 + 0x7FFF + ((w1 >> 16) & 1)) >> 16
    return r0 | (r1 << 16)


def _unpack_bf16_pair(w):
    c0 = lax.bitcast_convert_type(w << 16, jnp.float32)
    c1 = lax.bitcast_convert_type(w & jnp.uint32(0xFFFF0000), jnp.float32)
    return c0, c1


def _proj_body(x_ref, wq_ref, wk_ref, wv_ref, q_ref, k_ref, v_ref):
    x = x_ref[0]
    q_ref[0] = jnp.dot(x, wq_ref[...], preferred_element_type=jnp.float32)
    kk = jnp.dot(x, wk_ref[...], preferred_element_type=jnp.float32)
    vv = jnp.dot(x, wv_ref[...], preferred_element_type=jnp.float32)
    for t in range(NT):
        lo = slice(2 * t * 128, (2 * t + 1) * 128)
        hi = slice((2 * t + 1) * 128, (2 * t + 2) * 128)
        dst = slice(t * 128, (t + 1) * 128)
        k_ref[0, :, dst] = _pack_bf16_pair(kk[:, lo], kk[:, hi])
        v_ref[0, :, dst] = _pack_bf16_pair(vv[:, lo], vv[:, hi])


def _sc_gather_body(lseg, kp_hbm, vp_hbm, idx_hbm, kg_hbm, vg_hbm,
                    idx_v, kbuf, vbuf, ks0, ks1, vs0, vs1):
    nchunk = B * lseg // CH // NW
    ksem = (ks0, ks1)
    vsem = (vs0, vs1)
    wid = lax.axis_index("s") * 2 + lax.axis_index("c")
    row0 = wid * (B * lseg // NW)
    boff = (row0 // lseg) * LP
    chunk0 = wid * nchunk

    pltpu.sync_copy(idx_hbm.at[pl.ds(chunk0, nchunk)], idx_v)

    def adjust(c, carry):
        for h in range(IDX_CH // 16):
            sl = pl.ds(h * 16, 16)
            idx_v[c, sl] = idx_v[c, sl] + boff
        return carry

    lax.fori_loop(0, nchunk, adjust, 0)

    def fire(c, s):
        pltpu.async_copy(kp_hbm.at[idx_v.at[c]], kbuf.at[s], ksem[s])
        pltpu.async_copy(vp_hbm.at[idx_v.at[c]], vbuf.at[s], vsem[s])

    def drain(c, s):
        pltpu.make_async_copy(kp_hbm.at[idx_v.at[c]], kbuf.at[s],
                              ksem[s]).wait()
        pltpu.make_async_copy(vp_hbm.at[idx_v.at[c]], vbuf.at[s],
                              vsem[s]).wait()
        out_off = (chunk0 + c) * IDX_CH
        pltpu.sync_copy(kbuf.at[s], kg_hbm.at[pl.ds(out_off, IDX_CH)])
        pltpu.sync_copy(vbuf.at[s], vg_hbm.at[pl.ds(out_off, IDX_CH)])

    for s in range(NBUF):
        fire(s, s)

    def group(gi, carry):
        c = gi * NBUF
        for s in range(NBUF):
            drain(c + s, s)
            fire(c + NBUF + s, s)
        return carry

    lax.fori_loop(0, (nchunk - NBUF) // NBUF, group, 0)
    for s in range(NBUF):
        drain(nchunk - NBUF + s, s)


def _sj(j):
    d_over = (lax.broadcasted_iota(jnp.int32, (128, H), 0) + j * 128) // DH
    h_ids = lax.broadcasted_iota(jnp.int32, (128, H), 1)
    return (d_over == h_ids).astype(jnp.float32)


def _stj(j):
    d_over = (lax.broadcasted_iota(jnp.int32, (H, 128), 1) + j * 128) // DH
    h_ids = lax.broadcasted_iota(jnp.int32, (H, 128), 0)
    return (d_over == h_ids).astype(jnp.float32)


def _unpack_cols(ref, n):
    del n
    chunks = []
    for t in range(NT):
        w = ref[0, :, t * 128:(t + 1) * 128]
        c0, c1 = _unpack_bf16_pair(w)
        chunks.extend([c0, c1])
    return chunks


def _attn_body(q_ref, kg_ref, vg_ref, kself_ref, vself_ref, gk_ref, gv_ref,
               dist_ref, par_ref, wout_ref, bout_ref, o_ref):
    q = q_ref[0] * SCALE

    kg_c = _unpack_cols(kg_ref, TL * K)
    ks_c = _unpack_cols(kself_ref, TL)
    gk_c = _unpack_cols(gk_ref, G)

    S_cols = [_sj(j) for j in range(NJ)]
    St_cols = [_stj(j) for j in range(NJ)]

    def _gmask(j, g_c):
        mj = _stj(j)
        gexp = jnp.broadcast_to(g_c[j][:, None, :], (G, H, 128))
        return (gexp * mj[None, :, :]).reshape(G * H, 128)

    s_n = jnp.zeros((TL * K, H), jnp.float32)
    s_s = jnp.zeros((TL, H), jnp.float32)
    s_gf = jnp.zeros((TL, G * H), jnp.float32)
    dims = (((1,), (1,)), ((), ()))
    for j in range(NJ):
        qj = q[:, j * 128:(j + 1) * 128]
        qnj = jnp.broadcast_to(qj[:, None, :],
                               (TL, K, 128)).reshape(TL * K, 128)
        s_n += jnp.dot(qnj * kg_c[j], S_cols[j],
                       preferred_element_type=jnp.float32)
        s_s += jnp.dot(qj * ks_c[j], S_cols[j],
                       preferred_element_type=jnp.float32)
        s_gf += lax.dot_general(qj, _gmask(j, gk_c), dims,
                                preferred_element_type=jnp.float32)

    inv2s = par_ref[0:1, 0:H].reshape(1, 1, H)
    gbias = par_ref[1:2, 0:1]
    dist = dist_ref[0]
    s_n3 = s_n.reshape(TL, K, H) + (dist * dist)[:, :, None] * inv2s
    s_gf = s_gf + gbias

    m_g = ft.reduce(jnp.maximum,
                    [s_gf[:, g * H:(g + 1) * H] for g in range(G)])
    m = jnp.maximum(jnp.maximum(s_s, s_n3.max(axis=1)), m_g)
    e_s = jnp.exp(s_s - m)
    e_n = jnp.exp(s_n3 - m[:, None, :])
    m_t = jnp.concatenate([m] * G, axis=1)
    e_gf = jnp.exp(s_gf - m_t)
    sum_g = ft.reduce(jnp.add, [e_gf[:, g * H:(g + 1) * H] for g in range(G)])
    inv_den = 1.0 / (e_s + e_n.sum(axis=1) + sum_g)
    p_s = e_s * inv_den
    p_n = (e_n * inv_den[:, None, :]).reshape(TL * K, H)
    p_gf = e_gf * jnp.concatenate([inv_den] * G, axis=1)

    vg_c = _unpack_cols(vg_ref, TL * K)
    vs_c = _unpack_cols(vself_ref, TL)
    gv_c = _unpack_cols(gv_ref, G)

    outs = []
    for j in range(NJ):
        oj = jnp.dot(p_s, St_cols[j],
                     preferred_element_type=jnp.float32) * vs_c[j]
        pv = jnp.dot(p_n, St_cols[j],
                     preferred_element_type=jnp.float32) * vg_c[j]
        oj += pv.reshape(TL, K, 128).sum(axis=1)
        oj += jnp.dot(p_gf, _gmask(j, gv_c),
                      preferred_element_type=jnp.float32)
        outs.append(oj)
    out = jnp.concatenate(outs, axis=1)

    o_ref[0] = (jnp.dot(out, wout_ref[...], preferred_element_type=jnp.float32)
                + bout_ref[0:1, :])


@functools.cache
def _make_sc_gather(lseg):
    return pl.kernel(
        functools.partial(_sc_gather_body, lseg),
        mesh=plsc.VectorSubcoreMesh(core_axis_name="c", subcore_axis_name="s"),
        out_type=[jax.ShapeDtypeStruct((B * lseg * K, DP), jnp.uint32)] * 2,
        scratch_types=[
            pltpu.VMEM((B * lseg // CH // NW, IDX_CH), jnp.int32),
            pltpu.VMEM((NBUF, IDX_CH, DP), jnp.uint32),
            pltpu.VMEM((NBUF, IDX_CH, DP), jnp.uint32),
            pltpu.SemaphoreType.DMA,
            pltpu.SemaphoreType.DMA,
            pltpu.SemaphoreType.DMA,
            pltpu.SemaphoreType.DMA,
        ],
    )


def kernel(spatial, topk_indices, rpe, self_rpe, distances, global_latents,
           Wq, Wk, Wv, Wout, b_out, log_sigma, global_bias):
    xall = jnp.concatenate(
        [spatial, global_latents,
         jnp.zeros((B, LP - L - G, D), spatial.dtype)], axis=1)

    q_all, kp, vp = pl.pallas_call(
        _proj_body,
        grid=(B, LP // TLA),
        in_specs=[
            pl.BlockSpec((1, TLA, D), lambda b, i: (b, i, 0)),
            pl.BlockSpec((D, D), lambda b, i: (0, 0)),
            pl.BlockSpec((D, D), lambda b, i: (0, 0)),
            pl.BlockSpec((D, D), lambda b, i: (0, 0)),
        ],
        out_specs=[
            pl.BlockSpec((1, TLA, D), lambda b, i: (b, i, 0)),
            pl.BlockSpec((1, TLA, DP), lambda b, i: (b, i, 0)),
            pl.BlockSpec((1, TLA, DP), lambda b, i: (b, i, 0)),
        ],
        out_shape=[
            jax.ShapeDtypeStruct((B, LP, D), jnp.float32),
            jax.ShapeDtypeStruct((B, LP, DP), jnp.uint32),
            jax.ShapeDtypeStruct((B, LP, DP), jnp.uint32),
        ],
    )(xall, Wq, Wk, Wv)

    params = jnp.zeros((8, 128), jnp.float32)
    params = params.at[0, :H].set(-0.5 * jnp.exp(-2.0 * log_sigma))
    params = params.at[1, 0].set(global_bias)
    bout8 = jnp.broadcast_to(b_out[None, :], (8, D))
    gk = kp[:, L:L + G]
    gv = vp[:, L:L + G]
    kp_flat = kp.reshape(B * LP, DP)
    vp_flat = vp.reshape(B * LP, DP)
    idx32 = topk_indices.astype(jnp.int32)

    idx_chunks = idx32.reshape(B * L // CH, IDX_CH)
    kg_flat, vg_flat = _make_sc_gather(L)(kp_flat, vp_flat, idx_chunks)
    kg = kg_flat.reshape(B, L * K, DP)
    vg = vg_flat.reshape(B, L * K, DP)

    out = pl.pallas_call(
        _attn_body,
        grid=(B, L // TL),
        in_specs=[
            pl.BlockSpec((1, TL, D), lambda b, i: (b, i, 0)),
            pl.BlockSpec((1, TL * K, DP), lambda b, i: (b, i, 0)),
            pl.BlockSpec((1, TL * K, DP), lambda b, i: (b, i, 0)),
            pl.BlockSpec((1, TL, DP), lambda b, i: (b, i, 0)),
            pl.BlockSpec((1, TL, DP), lambda b, i: (b, i, 0)),
            pl.BlockSpec((1, G, DP), lambda b, i: (b, 0, 0)),
            pl.BlockSpec((1, G, DP), lambda b, i: (b, 0, 0)),
            pl.BlockSpec((1, TL, K), lambda b, i: (b, i, 0)),
            pl.BlockSpec((8, 128), lambda b, i: (0, 0)),
            pl.BlockSpec((D, D), lambda b, i: (0, 0)),
            pl.BlockSpec((8, D), lambda b, i: (0, 0)),
        ],
        out_specs=pl.BlockSpec((1, TL, D), lambda b, i: (b, i, 0)),
        out_shape=jax.ShapeDtypeStruct((B, L, D), jnp.float32),
    )(q_all[:, :L], kg, vg, kp[:, :L], vp[:, :L], gk, gv,
      distances, params, Wout, bout8)
    return out


# speedup vs baseline: 1.0680x; 1.0000x over previous
import functools
import functools as ft
import jax
import jax.numpy as jnp
from jax import lax
from jax.experimental import pallas as pl
from jax.experimental.pallas import tpu as pltpu
from jax.experimental.pallas import tpu_sc as plsc

H = 12
DH = 64
L = 2048
D = 768
K = 16
G = 8
B = 2
NJ = D // 128
NT = NJ // 2
DP = NT * 128
LP = 2176
TLA = 128
TL = 256
SCALE = DH ** -0.5

NW = 32
CH = 4
IDX_CH = CH * K
NBUF = 2


def _pack_bf16_pair(c0, c1):
    w0 = lax.bitcast_convert_type(c0, jnp.uint32)
    w1 = lax.bitcast_convert_type(c1, jnp.uint32)
    r0 = (w0 + 0x7FFF + ((w0 >> 16) & 1)) >> 16
    r1 = (w1 + 0x7FFF + ((w1 >> 16) & 1)) >> 16
    return r0 | (r1 << 16)


def _unpack_bf16_pair(w):
    c0 = lax.bitcast_convert_type(w << 16, jnp.float32)
    c1 = lax.bitcast_convert_type(w & jnp.uint32(0xFFFF0000), jnp.float32)
    return c0, c1


def _proj_body(x_ref, wq_ref, wk_ref, wv_ref, q_ref, k_ref, v_ref):
    x = x_ref[0]
    q_ref[0] = jnp.dot(x, wq_ref[...], preferred_element_type=jnp.float32)
    kk = jnp.dot(x, wk_ref[...], preferred_element_type=jnp.float32)
    vv = jnp.dot(x, wv_ref[...], preferred_element_type=jnp.float32)
    for t in range(NT):
        lo = slice(2 * t * 128, (2 * t + 1) * 128)
        hi = slice((2 * t + 1) * 128, (2 * t + 2) * 128)
        dst = slice(t * 128, (t + 1) * 128)
        k_ref[0, :, dst] = _pack_bf16_pair(kk[:, lo], kk[:, hi])
        v_ref[0, :, dst] = _pack_bf16_pair(vv[:, lo], vv[:, hi])


def _sc_gather_body(lseg, kp_hbm, vp_hbm, idx_hbm, kg_hbm, vg_hbm,
                    idx_v, kbuf, vbuf, ks0, ks1, vs0, vs1):
    nchunk = B * lseg // CH // NW
    ksem = (ks0, ks1)
    vsem = (vs0, vs1)
    wid = lax.axis_index("s") * 2 + lax.axis_index("c")
    row0 = wid * (B * lseg // NW)
    boff = (row0 // lseg) * LP
    chunk0 = wid * nchunk

    pltpu.sync_copy(idx_hbm.at[pl.ds(chunk0, nchunk)], idx_v)

    def adjust(c, carry):
        for h in range(IDX_CH // 16):
            sl = pl.ds(h * 16, 16)
            idx_v[c, sl] = idx_v[c, sl] + boff
        return carry

    lax.fori_loop(0, nchunk, adjust, 0)

    def fire(c, s):
        pltpu.async_copy(kp_hbm.at[idx_v.at[c]], kbuf.at[s], ksem[s])
        pltpu.async_copy(vp_hbm.at[idx_v.at[c]], vbuf.at[s], vsem[s])

    def drain(c, s):
        pltpu.make_async_copy(kp_hbm.at[idx_v.at[c]], kbuf.at[s],
                              ksem[s]).wait()
        pltpu.make_async_copy(vp_hbm.at[idx_v.at[c]], vbuf.at[s],
                              vsem[s]).wait()
        out_off = (chunk0 + c) * IDX_CH
        pltpu.sync_copy(kbuf.at[s], kg_hbm.at[pl.ds(out_off, IDX_CH)])
        pltpu.sync_copy(vbuf.at[s], vg_hbm.at[pl.ds(out_off, IDX_CH)])

    for s in range(NBUF):
        fire(s, s)

    def group(gi, carry):
        c = gi * NBUF
        for s in range(NBUF):
            drain(c + s, s)
            fire(c + NBUF + s, s)
        return carry

    lax.fori_loop(0, (nchunk - NBUF) // NBUF, group, 0)
    for s in range(NBUF):
        drain(nchunk - NBUF + s, s)


def _sj(j):
    d_over = (lax.broadcasted_iota(jnp.int32, (128, H), 0) + j * 128) // DH
    h_ids = lax.broadcasted_iota(jnp.int32, (128, H), 1)
    return (d_over == h_ids).astype(jnp.float32)


def _stj(j):
    d_over = (lax.broadcasted_iota(jnp.int32, (H, 128), 1) + j * 128) // DH
    h_ids = lax.broadcasted_iota(jnp.int32, (H, 128), 0)
    return (d_over == h_ids).astype(jnp.float32)


def _unpack_cols(ref, n):
    del n
    chunks = []
    for t in range(NT):
        w = ref[0, :, t * 128:(t + 1) * 128]
        c0, c1 = _unpack_bf16_pair(w)
        chunks.extend([c0, c1])
    return chunks


def _attn_body(q_ref, kg_ref, vg_ref, kself_ref, vself_ref, gk_ref, gv_ref,
               dist_ref, par_ref, wout_ref, bout_ref, o_ref):
    q = q_ref[0] * SCALE

    kg_c = _unpack_cols(kg_ref, TL * K)
    ks_c = _unpack_cols(kself_ref, TL)
    gk_c = _unpack_cols(gk_ref, G)

    S_cols = [_sj(j) for j in range(NJ)]
    St_cols = [_stj(j) for j in range(NJ)]

    def _gmask(j, g_c):
        mj = _stj(j)
        gexp = jnp.broadcast_to(g_c[j][:, None, :], (G, H, 128))
        return (gexp * mj[None, :, :]).reshape(G * H, 128)

    s_n = jnp.zeros((TL * K, H), jnp.float32)
    s_s = jnp.zeros((TL, H), jnp.float32)
    s_gf = jnp.zeros((TL, G * H), jnp.float32)
    dims = (((1,), (1,)), ((), ()))
    for j in range(NJ):
        qj = q[:, j * 128:(j + 1) * 128]
        qnj = jnp.broadcast_to(qj[:, None, :],
                               (TL, K, 128)).reshape(TL * K, 128)
        s_n += jnp.dot(qnj * kg_c[j], S_cols[j],
                       preferred_element_type=jnp.float32)
        s_s += jnp.dot(qj * ks_c[j], S_cols[j],
                       preferred_element_type=jnp.float32)
        s_gf += lax.dot_general(qj, _gmask(j, gk_c), dims,
                                preferred_element_type=jnp.float32)

    inv2s = par_ref[0:1, 0:H].reshape(1, 1, H)
    gbias = par_ref[1:2, 0:1]
    dist = dist_ref[0]
    s_n3 = s_n.reshape(TL, K, H) + (dist * dist)[:, :, None] * inv2s
    s_gf = s_gf + gbias

    m_g = ft.reduce(jnp.maximum,
                    [s_gf[:, g * H:(g + 1) * H] for g in range(G)])
    m = jnp.maximum(jnp.maximum(s_s, s_n3.max(axis=1)), m_g)
    e_s = jnp.exp(s_s - m)
    e_n = jnp.exp(s_n3 - m[:, None, :])
    m_t = jnp.concatenate([m] * G, axis=1)
    e_gf = jnp.exp(s_gf - m_t)
    sum_g = ft.reduce(jnp.add, [e_gf[:, g * H:(g + 1) * H] for g in range(G)])
    inv_den = 1.0 / (e_s + e_n.sum(axis=1) + sum_g)
    p_s = e_s * inv_den
    p_n = (e_n * inv_den[:, None, :]).reshape(TL * K, H)
    p_gf = e_gf * jnp.concatenate([inv_den] * G, axis=1)

    vg_c = _unpack_cols(vg_ref, TL * K)
    vs_c = _unpack_cols(vself_ref, TL)
    gv_c = _unpack_cols(gv_ref, G)

    outs = []
    for j in range(NJ):
        oj = jnp.dot(p_s, St_cols[j],
                     preferred_element_type=jnp.float32) * vs_c[j]
        pv = jnp.dot(p_n, St_cols[j],
                     preferred_element_type=jnp.float32) * vg_c[j]
        oj += pv.reshape(TL, K, 128).sum(axis=1)
        oj += jnp.dot(p_gf, _gmask(j, gv_c),
                      preferred_element_type=jnp.float32)
        outs.append(oj)
    out = jnp.concatenate(outs, axis=1)

    o_ref[0] = (jnp.dot(out, wout_ref[...], preferred_element_type=jnp.float32)
                + bout_ref[0:1, :])


@functools.cache
def _make_sc_gather(lseg):
    return pl.kernel(
        functools.partial(_sc_gather_body, lseg),
        mesh=plsc.VectorSubcoreMesh(core_axis_name="c", subcore_axis_name="s"),
        out_type=[jax.ShapeDtypeStruct((B * lseg * K, DP), jnp.uint32)] * 2,
        scratch_types=[
            pltpu.VMEM((B * lseg // CH // NW, IDX_CH), jnp.int32),
            pltpu.VMEM((NBUF, IDX_CH, DP), jnp.uint32),
            pltpu.VMEM((NBUF, IDX_CH, DP), jnp.uint32),
            pltpu.SemaphoreType.DMA,
            pltpu.SemaphoreType.DMA,
            pltpu.SemaphoreType.DMA,
            pltpu.SemaphoreType.DMA,
        ],
    )


def kernel(spatial, topk_indices, rpe, self_rpe, distances, global_latents,
           Wq, Wk, Wv, Wout, b_out, log_sigma, global_bias):
    xall = jnp.concatenate(
        [spatial, global_latents,
         jnp.zeros((B, LP - L - G, D), spatial.dtype)], axis=1)

    q_all, kp, vp = pl.pallas_call(
        _proj_body,
        grid=(B, LP // TLA),
        in_specs=[
            pl.BlockSpec((1, TLA, D), lambda b, i: (b, i, 0)),
            pl.BlockSpec((D, D), lambda b, i: (0, 0)),
            pl.BlockSpec((D, D), lambda b, i: (0, 0)),
            pl.BlockSpec((D, D), lambda b, i: (0, 0)),
        ],
        out_specs=[
            pl.BlockSpec((1, TLA, D), lambda b, i: (b, i, 0)),
            pl.BlockSpec((1, TLA, DP), lambda b, i: (b, i, 0)),
            pl.BlockSpec((1, TLA, DP), lambda b, i: (b, i, 0)),
        ],
        out_shape=[
            jax.ShapeDtypeStruct((B, LP, D), jnp.float32),
            jax.ShapeDtypeStruct((B, LP, DP), jnp.uint32),
            jax.ShapeDtypeStruct((B, LP, DP), jnp.uint32),
        ],
    )(xall, Wq, Wk, Wv)

    params = jnp.zeros((8, 128), jnp.float32)
    params = params.at[0, :H].set(-0.5 * jnp.exp(-2.0 * log_sigma))
    params = params.at[1, 0].set(global_bias)
    bout8 = jnp.broadcast_to(b_out[None, :], (8, D))
    gk = kp[:, L:L + G]
    gv = vp[:, L:L + G]
    kp_flat = kp.reshape(B * LP, DP)
    vp_flat = vp.reshape(B * LP, DP)
    idx32 = topk_indices.astype(jnp.int32)

    idx_chunks = idx32.reshape(B * L // CH, IDX_CH)
    kg_flat, vg_flat = _make_sc_gather(L)(kp_flat, vp_flat, idx_chunks)
    kg = kg_flat.reshape(B, L * K, DP)
    vg = vg_flat.reshape(B, L * K, DP)

    out = pl.pallas_call(
        _attn_body,
        grid=(B, L // TL),
        in_specs=[
            pl.BlockSpec((1, TL, D), lambda b, i: (b, i, 0)),
            pl.BlockSpec((1, TL * K, DP), lambda b, i: (b, i, 0)),
            pl.BlockSpec((1, TL * K, DP), lambda b, i: (b, i, 0)),
            pl.BlockSpec((1, TL, DP), lambda b, i: (b, i, 0)),
            pl.BlockSpec((1, TL, DP), lambda b, i: (b, i, 0)),
            pl.BlockSpec((1, G, DP), lambda b, i: (b, 0, 0)),
            pl.BlockSpec((1, G, DP), lambda b, i: (b, 0, 0)),
            pl.BlockSpec((1, TL, K), lambda b, i: (b, i, 0)),
            pl.BlockSpec((8, 128), lambda b, i: (0, 0)),
            pl.BlockSpec((D, D), lambda b, i: (0, 0)),
            pl.BlockSpec((8, D), lambda b, i: (0, 0)),
        ],
        out_specs=pl.BlockSpec((1, TL, D), lambda b, i: (b, i, 0)),
        out_shape=jax.ShapeDtypeStruct((B, L, D), jnp.float32),
    )(q_all[:, :L], kg, vg, kp[:, :L], vp[:, :L], gk, gv,
      distances, params, Wout, bout8)
    return out


# speedup vs baseline: 1.1319x; 1.0599x over previous
import functools
import functools as ft
import jax
import jax.numpy as jnp
from jax import lax
from jax.experimental import pallas as pl
from jax.experimental.pallas import tpu as pltpu
from jax.experimental.pallas import tpu_sc as plsc

H = 12
DH = 64
L = 2048
D = 768
K = 16
G = 8
B = 2
NJ = D // 128
NT = NJ // 2
DP = NT * 128
LP = 2176
TLA = 128
TL = 128
SCALE = DH ** -0.5

NW = 32
CH = 4
IDX_CH = CH * K
NBUF = 2


def _pack_bf16_pair(c0, c1):
    w0 = lax.bitcast_convert_type(c0, jnp.uint32)
    w1 = lax.bitcast_convert_type(c1, jnp.uint32)
    r0 = (w0 + 0x7FFF + ((w0 >> 16) & 1)) >> 16
    r1 = (w1 + 0x7FFF + ((w1 >> 16) & 1)) >> 16
    return r0 | (r1 << 16)


def _unpack_bf16_pair(w):
    c0 = lax.bitcast_convert_type(w << 16, jnp.float32)
    c1 = lax.bitcast_convert_type(w & jnp.uint32(0xFFFF0000), jnp.float32)
    return c0, c1


def _proj_body(x_ref, wq_ref, wk_ref, wv_ref, q_ref, k_ref, v_ref):
    x = x_ref[0]
    q_ref[0] = jnp.dot(x, wq_ref[...], preferred_element_type=jnp.float32)
    kk = jnp.dot(x, wk_ref[...], preferred_element_type=jnp.float32)
    vv = jnp.dot(x, wv_ref[...], preferred_element_type=jnp.float32)
    for t in range(NT):
        lo = slice(2 * t * 128, (2 * t + 1) * 128)
        hi = slice((2 * t + 1) * 128, (2 * t + 2) * 128)
        dst = slice(t * 128, (t + 1) * 128)
        k_ref[0, :, dst] = _pack_bf16_pair(kk[:, lo], kk[:, hi])
        v_ref[0, :, dst] = _pack_bf16_pair(vv[:, lo], vv[:, hi])


def _sc_gather_body(lseg, kp_hbm, vp_hbm, idx_hbm, kg_hbm, vg_hbm,
                    idx_v, kbuf, vbuf, ks0, ks1, vs0, vs1):
    nchunk = B * lseg // CH // NW
    ksem = (ks0, ks1)
    vsem = (vs0, vs1)
    wid = lax.axis_index("s") * 2 + lax.axis_index("c")
    row0 = wid * (B * lseg // NW)
    boff = (row0 // lseg) * LP
    chunk0 = wid * nchunk

    pltpu.sync_copy(idx_hbm.at[pl.ds(chunk0, nchunk)], idx_v)

    def adjust(c, carry):
        for h in range(IDX_CH // 16):
            sl = pl.ds(h * 16, 16)
            idx_v[c, sl] = idx_v[c, sl] + boff
        return carry

    lax.fori_loop(0, nchunk, adjust, 0)

    def fire(c, s):
        pltpu.async_copy(kp_hbm.at[idx_v.at[c]], kbuf.at[s], ksem[s])
        pltpu.async_copy(vp_hbm.at[idx_v.at[c]], vbuf.at[s], vsem[s])

    def drain(c, s):
        pltpu.make_async_copy(kp_hbm.at[idx_v.at[c]], kbuf.at[s],
                              ksem[s]).wait()
        pltpu.make_async_copy(vp_hbm.at[idx_v.at[c]], vbuf.at[s],
                              vsem[s]).wait()
        out_off = (chunk0 + c) * IDX_CH
        pltpu.sync_copy(kbuf.at[s], kg_hbm.at[pl.ds(out_off, IDX_CH)])
        pltpu.sync_copy(vbuf.at[s], vg_hbm.at[pl.ds(out_off, IDX_CH)])

    for s in range(NBUF):
        fire(s, s)

    def group(gi, carry):
        c = gi * NBUF
        for s in range(NBUF):
            drain(c + s, s)
            fire(c + NBUF + s, s)
        return carry

    lax.fori_loop(0, (nchunk - NBUF) // NBUF, group, 0)
    for s in range(NBUF):
        drain(nchunk - NBUF + s, s)


def _sj(j):
    d_over = (lax.broadcasted_iota(jnp.int32, (128, H), 0) + j * 128) // DH
    h_ids = lax.broadcasted_iota(jnp.int32, (128, H), 1)
    return (d_over == h_ids).astype(jnp.float32)


def _stj(j):
    d_over = (lax.broadcasted_iota(jnp.int32, (H, 128), 1) + j * 128) // DH
    h_ids = lax.broadcasted_iota(jnp.int32, (H, 128), 0)
    return (d_over == h_ids).astype(jnp.float32)


def _unpack_cols(ref, n):
    del n
    chunks = []
    for t in range(NT):
        w = ref[0, :, t * 128:(t + 1) * 128]
        c0, c1 = _unpack_bf16_pair(w)
        chunks.extend([c0, c1])
    return chunks


def _attn_body(q_ref, kg_ref, vg_ref, kself_ref, vself_ref, gk_ref, gv_ref,
               dist_ref, par_ref, wout_ref, bout_ref, o_ref):
    q = q_ref[0] * SCALE

    kg_c = _unpack_cols(kg_ref, TL * K)
    ks_c = _unpack_cols(kself_ref, TL)
    gk_c = _unpack_cols(gk_ref, G)

    d_over = lax.broadcasted_iota(jnp.int32, (D, H), 0) // DH
    h_ids = lax.broadcasted_iota(jnp.int32, (D, H), 1)
    S = (d_over == h_ids).astype(jnp.float32)
    d_over_t = lax.broadcasted_iota(jnp.int32, (H, D), 1) // DH
    h_ids_t = lax.broadcasted_iota(jnp.int32, (H, D), 0)
    St = (d_over_t == h_ids_t).astype(jnp.float32)

    def _gmask(g_c):
        grow = jnp.concatenate(g_c, axis=1)
        gexp = jnp.broadcast_to(grow[:, None, :], (G, H, D))
        return (gexp * St[None, :, :]).reshape(G * H, D)

    kg = jnp.concatenate(kg_c, axis=1)
    ks = jnp.concatenate(ks_c, axis=1)
    qn = jnp.broadcast_to(q[:, None, :], (TL, K, D)).reshape(TL * K, D)
    dims = (((1,), (1,)), ((), ()))
    s_n = jnp.dot(qn * kg, S, preferred_element_type=jnp.float32)
    s_s = jnp.dot(q * ks, S, preferred_element_type=jnp.float32)
    s_gf = lax.dot_general(q, _gmask(gk_c), dims,
                           preferred_element_type=jnp.float32)

    inv2s = par_ref[0:1, 0:H].reshape(1, 1, H)
    gbias = par_ref[1:2, 0:1]
    dist = dist_ref[0]
    s_n3 = s_n.reshape(TL, K, H) + (dist * dist)[:, :, None] * inv2s
    s_gf = s_gf + gbias

    m_g = ft.reduce(jnp.maximum,
                    [s_gf[:, g * H:(g + 1) * H] for g in range(G)])
    m = jnp.maximum(jnp.maximum(s_s, s_n3.max(axis=1)), m_g)
    e_s = jnp.exp(s_s - m)
    e_n = jnp.exp(s_n3 - m[:, None, :])
    m_t = jnp.concatenate([m] * G, axis=1)
    e_gf = jnp.exp(s_gf - m_t)
    sum_g = ft.reduce(jnp.add, [e_gf[:, g * H:(g + 1) * H] for g in range(G)])
    inv_den = 1.0 / (e_s + e_n.sum(axis=1) + sum_g)
    p_s = e_s * inv_den
    p_n = (e_n * inv_den[:, None, :]).reshape(TL * K, H)
    p_gf = e_gf * jnp.concatenate([inv_den] * G, axis=1)

    vg_c = _unpack_cols(vg_ref, TL * K)
    vs_c = _unpack_cols(vself_ref, TL)
    gv_c = _unpack_cols(gv_ref, G)

    vg = jnp.concatenate(vg_c, axis=1)
    vs = jnp.concatenate(vs_c, axis=1)
    out = jnp.dot(p_s, St, preferred_element_type=jnp.float32) * vs
    pv = jnp.dot(p_n, St, preferred_element_type=jnp.float32) * vg
    out += pv.reshape(TL, K, D).sum(axis=1)
    out += jnp.dot(p_gf, _gmask(gv_c), preferred_element_type=jnp.float32)

    o_ref[0] = (jnp.dot(out, wout_ref[...], preferred_element_type=jnp.float32)
                + bout_ref[0:1, :])


@functools.cache
def _make_sc_gather(lseg):
    return pl.kernel(
        functools.partial(_sc_gather_body, lseg),
        mesh=plsc.VectorSubcoreMesh(core_axis_name="c", subcore_axis_name="s"),
        out_type=[jax.ShapeDtypeStruct((B * lseg * K, DP), jnp.uint32)] * 2,
        scratch_types=[
            pltpu.VMEM((B * lseg // CH // NW, IDX_CH), jnp.int32),
            pltpu.VMEM((NBUF, IDX_CH, DP), jnp.uint32),
            pltpu.VMEM((NBUF, IDX_CH, DP), jnp.uint32),
            pltpu.SemaphoreType.DMA,
            pltpu.SemaphoreType.DMA,
            pltpu.SemaphoreType.DMA,
            pltpu.SemaphoreType.DMA,
        ],
    )


def kernel(spatial, topk_indices, rpe, self_rpe, distances, global_latents,
           Wq, Wk, Wv, Wout, b_out, log_sigma, global_bias):
    xall = jnp.concatenate(
        [spatial, global_latents,
         jnp.zeros((B, LP - L - G, D), spatial.dtype)], axis=1)

    q_all, kp, vp = pl.pallas_call(
        _proj_body,
        grid=(B, LP // TLA),
        in_specs=[
            pl.BlockSpec((1, TLA, D), lambda b, i: (b, i, 0)),
            pl.BlockSpec((D, D), lambda b, i: (0, 0)),
            pl.BlockSpec((D, D), lambda b, i: (0, 0)),
            pl.BlockSpec((D, D), lambda b, i: (0, 0)),
        ],
        out_specs=[
            pl.BlockSpec((1, TLA, D), lambda b, i: (b, i, 0)),
            pl.BlockSpec((1, TLA, DP), lambda b, i: (b, i, 0)),
            pl.BlockSpec((1, TLA, DP), lambda b, i: (b, i, 0)),
        ],
        out_shape=[
            jax.ShapeDtypeStruct((B, LP, D), jnp.float32),
            jax.ShapeDtypeStruct((B, LP, DP), jnp.uint32),
            jax.ShapeDtypeStruct((B, LP, DP), jnp.uint32),
        ],
    )(xall, Wq, Wk, Wv)

    params = jnp.zeros((8, 128), jnp.float32)
    params = params.at[0, :H].set(-0.5 * jnp.exp(-2.0 * log_sigma))
    params = params.at[1, 0].set(global_bias)
    bout8 = jnp.broadcast_to(b_out[None, :], (8, D))
    gk = kp[:, L:L + G]
    gv = vp[:, L:L + G]
    kp_flat = kp.reshape(B * LP, DP)
    vp_flat = vp.reshape(B * LP, DP)
    idx32 = topk_indices.astype(jnp.int32)

    idx_chunks = idx32.reshape(B * L // CH, IDX_CH)
    kg_flat, vg_flat = _make_sc_gather(L)(kp_flat, vp_flat, idx_chunks)
    kg = kg_flat.reshape(B, L * K, DP)
    vg = vg_flat.reshape(B, L * K, DP)

    out = pl.pallas_call(
        _attn_body,
        grid=(B, L // TL),
        in_specs=[
            pl.BlockSpec((1, TL, D), lambda b, i: (b, i, 0)),
            pl.BlockSpec((1, TL * K, DP), lambda b, i: (b, i, 0)),
            pl.BlockSpec((1, TL * K, DP), lambda b, i: (b, i, 0)),
            pl.BlockSpec((1, TL, DP), lambda b, i: (b, i, 0)),
            pl.BlockSpec((1, TL, DP), lambda b, i: (b, i, 0)),
            pl.BlockSpec((1, G, DP), lambda b, i: (b, 0, 0)),
            pl.BlockSpec((1, G, DP), lambda b, i: (b, 0, 0)),
            pl.BlockSpec((1, TL, K), lambda b, i: (b, i, 0)),
            pl.BlockSpec((8, 128), lambda b, i: (0, 0)),
            pl.BlockSpec((D, D), lambda b, i: (0, 0)),
            pl.BlockSpec((8, D), lambda b, i: (0, 0)),
        ],
        out_specs=pl.BlockSpec((1, TL, D), lambda b, i: (b, i, 0)),
        out_shape=jax.ShapeDtypeStruct((B, L, D), jnp.float32),
    )(q_all[:, :L], kg, vg, kp[:, :L], vp[:, :L], gk, gv,
      distances, params, Wout, bout8)
    return out


# speedup vs baseline: 1.1411x; 1.0081x over previous
import functools
import functools as ft
import jax
import jax.numpy as jnp
from jax import lax
from jax.experimental import pallas as pl
from jax.experimental.pallas import tpu as pltpu
from jax.experimental.pallas import tpu_sc as plsc

H = 12
DH = 64
L = 2048
D = 768
K = 16
G = 8
B = 2
NJ = D // 128
NT = NJ // 2
DP = NT * 128
LP = 2176
TLA = 128
TL = 256
SCALE = DH ** -0.5

NW = 32
CH = 4
IDX_CH = CH * K
NBUF = 2


def _pack_bf16_pair(c0, c1):
    w0 = lax.bitcast_convert_type(c0, jnp.uint32)
    w1 = lax.bitcast_convert_type(c1, jnp.uint32)
    r0 = (w0 + 0x7FFF + ((w0 >> 16) & 1)) >> 16
    r1 = (w1 + 0x7FFF + ((w1 >> 16) & 1)) >> 16
    return r0 | (r1 << 16)


def _unpack_bf16_pair(w):
    c0 = lax.bitcast_convert_type(w << 16, jnp.float32)
    c1 = lax.bitcast_convert_type(w & jnp.uint32(0xFFFF0000), jnp.float32)
    return c0, c1


def _proj_body(x_ref, wq_ref, wk_ref, wv_ref, q_ref, k_ref, v_ref):
    x = x_ref[0]
    q_ref[0] = jnp.dot(x, wq_ref[...], preferred_element_type=jnp.float32)
    kk = jnp.dot(x, wk_ref[...], preferred_element_type=jnp.float32)
    vv = jnp.dot(x, wv_ref[...], preferred_element_type=jnp.float32)
    for t in range(NT):
        lo = slice(2 * t * 128, (2 * t + 1) * 128)
        hi = slice((2 * t + 1) * 128, (2 * t + 2) * 128)
        dst = slice(t * 128, (t + 1) * 128)
        k_ref[0, :, dst] = _pack_bf16_pair(kk[:, lo], kk[:, hi])
        v_ref[0, :, dst] = _pack_bf16_pair(vv[:, lo], vv[:, hi])


def _sc_gather_body(lseg, kp_hbm, vp_hbm, idx_hbm, kg_hbm, vg_hbm,
                    idx_v, kbuf, vbuf, ks0, ks1, vs0, vs1):
    nchunk = B * lseg // CH // NW
    ksem = (ks0, ks1)
    vsem = (vs0, vs1)
    wid = lax.axis_index("s") * 2 + lax.axis_index("c")
    row0 = wid * (B * lseg // NW)
    boff = (row0 // lseg) * LP
    chunk0 = wid * nchunk

    pltpu.sync_copy(idx_hbm.at[pl.ds(chunk0, nchunk)], idx_v)

    def adjust(c, carry):
        for h in range(IDX_CH // 16):
            sl = pl.ds(h * 16, 16)
            idx_v[c, sl] = idx_v[c, sl] + boff
        return carry

    lax.fori_loop(0, nchunk, adjust, 0)

    def fire(c, s):
        pltpu.async_copy(kp_hbm.at[idx_v.at[c]], kbuf.at[s], ksem[s])
        pltpu.async_copy(vp_hbm.at[idx_v.at[c]], vbuf.at[s], vsem[s])

    def drain(c, s):
        pltpu.make_async_copy(kp_hbm.at[idx_v.at[c]], kbuf.at[s],
                              ksem[s]).wait()
        pltpu.make_async_copy(vp_hbm.at[idx_v.at[c]], vbuf.at[s],
                              vsem[s]).wait()
        out_off = (chunk0 + c) * IDX_CH
        pltpu.sync_copy(kbuf.at[s], kg_hbm.at[pl.ds(out_off, IDX_CH)])
        pltpu.sync_copy(vbuf.at[s], vg_hbm.at[pl.ds(out_off, IDX_CH)])

    for s in range(NBUF):
        fire(s, s)

    def group(gi, carry):
        c = gi * NBUF
        for s in range(NBUF):
            drain(c + s, s)
            fire(c + NBUF + s, s)
        return carry

    lax.fori_loop(0, (nchunk - NBUF) // NBUF, group, 0)
    for s in range(NBUF):
        drain(nchunk - NBUF + s, s)


def _unpack_cols(ref):
    chunks = []
    for t in range(NT):
        w = ref[0, :, t * 128:(t + 1) * 128]
        c0, c1 = _unpack_bf16_pair(w)
        chunks.extend([c0, c1])
    return chunks


def _attn_body(q_ref, kg_ref, vg_ref, kself_ref, vself_ref, gk_ref, gv_ref,
               dist_ref, par_ref, wout_ref, bout_ref, o_ref):
    q = q_ref[0] * SCALE

    kg_c = _unpack_cols(kg_ref)
    ks_c = _unpack_cols(kself_ref)
    gk_c = _unpack_cols(gk_ref)

    d_over = lax.broadcasted_iota(jnp.int32, (D, H), 0) // DH
    h_ids = lax.broadcasted_iota(jnp.int32, (D, H), 1)
    S = (d_over == h_ids).astype(jnp.float32)
    d_over_t = lax.broadcasted_iota(jnp.int32, (H, D), 1) // DH
    h_ids_t = lax.broadcasted_iota(jnp.int32, (H, D), 0)
    St = (d_over_t == h_ids_t).astype(jnp.float32)

    def _gmask(g_c):
        grow = jnp.concatenate(g_c, axis=1)
        gexp = jnp.broadcast_to(grow[:, None, :], (G, H, D))
        return (gexp * St[None, :, :]).reshape(G * H, D)

    kg = jnp.concatenate(kg_c, axis=1)
    ks = jnp.concatenate(ks_c, axis=1)
    qn = jnp.broadcast_to(q[:, None, :], (TL, K, D)).reshape(TL * K, D)
    dims = (((1,), (1,)), ((), ()))
    s_n = jnp.dot(qn * kg, S, preferred_element_type=jnp.float32)
    s_s = jnp.dot(q * ks, S, preferred_element_type=jnp.float32)
    s_gf = lax.dot_general(q, _gmask(gk_c), dims,
                           preferred_element_type=jnp.float32)

    inv2s = par_ref[0:1, 0:H].reshape(1, 1, H)
    gbias = par_ref[1:2, 0:1]
    dist = dist_ref[0]
    s_n3 = s_n.reshape(TL, K, H) + (dist * dist)[:, :, None] * inv2s
    s_gf = s_gf + gbias

    m_g = ft.reduce(jnp.maximum,
                    [s_gf[:, g * H:(g + 1) * H] for g in range(G)])
    m = jnp.maximum(jnp.maximum(s_s, s_n3.max(axis=1)), m_g)
    e_s = jnp.exp(s_s - m)
    e_n = jnp.exp(s_n3 - m[:, None, :])
    m_t = jnp.concatenate([m] * G, axis=1)
    e_gf = jnp.exp(s_gf - m_t)
    sum_g = ft.reduce(jnp.add, [e_gf[:, g * H:(g + 1) * H] for g in range(G)])
    inv_den = 1.0 / (e_s + e_n.sum(axis=1) + sum_g)
    p_s = e_s * inv_den
    p_n = (e_n * inv_den[:, None, :]).reshape(TL * K, H)
    p_gf = e_gf * jnp.concatenate([inv_den] * G, axis=1)

    vg_c = _unpack_cols(vg_ref)
    vs_c = _unpack_cols(vself_ref)
    gv_c = _unpack_cols(gv_ref)

    vg = jnp.concatenate(vg_c, axis=1)
    vs = jnp.concatenate(vs_c, axis=1)
    out = jnp.dot(p_s, St, preferred_element_type=jnp.float32) * vs
    pv = jnp.dot(p_n, St, preferred_element_type=jnp.float32) * vg
    out += pv.reshape(TL, K, D).sum(axis=1)
    out += jnp.dot(p_gf, _gmask(gv_c), preferred_element_type=jnp.float32)

    o_ref[0] = (jnp.dot(out, wout_ref[...], preferred_element_type=jnp.float32)
                + bout_ref[0:1, :])


@functools.cache
def _make_sc_gather(lseg):
    return pl.kernel(
        functools.partial(_sc_gather_body, lseg),
        mesh=plsc.VectorSubcoreMesh(core_axis_name="c", subcore_axis_name="s"),
        out_type=[jax.ShapeDtypeStruct((B * lseg * K, DP), jnp.uint32)] * 2,
        scratch_types=[
            pltpu.VMEM((B * lseg // CH // NW, IDX_CH), jnp.int32),
            pltpu.VMEM((NBUF, IDX_CH, DP), jnp.uint32),
            pltpu.VMEM((NBUF, IDX_CH, DP), jnp.uint32),
            pltpu.SemaphoreType.DMA,
            pltpu.SemaphoreType.DMA,
            pltpu.SemaphoreType.DMA,
            pltpu.SemaphoreType.DMA,
        ],
    )


def kernel(spatial, topk_indices, rpe, self_rpe, distances, global_latents,
           Wq, Wk, Wv, Wout, b_out, log_sigma, global_bias):
    xall = jnp.concatenate(
        [spatial, global_latents,
         jnp.zeros((B, LP - L - G, D), spatial.dtype)], axis=1)

    q_all, kp, vp = pl.pallas_call(
        _proj_body,
        grid=(B, LP // TLA),
        in_specs=[
            pl.BlockSpec((1, TLA, D), lambda b, i: (b, i, 0)),
            pl.BlockSpec((D, D), lambda b, i: (0, 0)),
            pl.BlockSpec((D, D), lambda b, i: (0, 0)),
            pl.BlockSpec((D, D), lambda b, i: (0, 0)),
        ],
        out_specs=[
            pl.BlockSpec((1, TLA, D), lambda b, i: (b, i, 0)),
            pl.BlockSpec((1, TLA, DP), lambda b, i: (b, i, 0)),
            pl.BlockSpec((1, TLA, DP), lambda b, i: (b, i, 0)),
        ],
        out_shape=[
            jax.ShapeDtypeStruct((B, LP, D), jnp.float32),
            jax.ShapeDtypeStruct((B, LP, DP), jnp.uint32),
            jax.ShapeDtypeStruct((B, LP, DP), jnp.uint32),
        ],
    )(xall, Wq, Wk, Wv)

    params = jnp.zeros((8, 128), jnp.float32)
    params = params.at[0, :H].set(-0.5 * jnp.exp(-2.0 * log_sigma))
    params = params.at[1, 0].set(global_bias)
    bout8 = jnp.broadcast_to(b_out[None, :], (8, D))
    gk = kp[:, L:L + G]
    gv = vp[:, L:L + G]
    kp_flat = kp.reshape(B * LP, DP)
    vp_flat = vp.reshape(B * LP, DP)
    idx32 = topk_indices.astype(jnp.int32)

    idx_chunks = idx32.reshape(B * L // CH, IDX_CH)
    kg_flat, vg_flat = _make_sc_gather(L)(kp_flat, vp_flat, idx_chunks)
    kg = kg_flat.reshape(B, L * K, DP)
    vg = vg_flat.reshape(B, L * K, DP)

    out = pl.pallas_call(
        _attn_body,
        grid=(B, L // TL),
        in_specs=[
            pl.BlockSpec((1, TL, D), lambda b, i: (b, i, 0)),
            pl.BlockSpec((1, TL * K, DP), lambda b, i: (b, i, 0)),
            pl.BlockSpec((1, TL * K, DP), lambda b, i: (b, i, 0)),
            pl.BlockSpec((1, TL, DP), lambda b, i: (b, i, 0)),
            pl.BlockSpec((1, TL, DP), lambda b, i: (b, i, 0)),
            pl.BlockSpec((1, G, DP), lambda b, i: (b, 0, 0)),
            pl.BlockSpec((1, G, DP), lambda b, i: (b, 0, 0)),
            pl.BlockSpec((1, TL, K), lambda b, i: (b, i, 0)),
            pl.BlockSpec((8, 128), lambda b, i: (0, 0)),
            pl.BlockSpec((D, D), lambda b, i: (0, 0)),
            pl.BlockSpec((8, D), lambda b, i: (0, 0)),
        ],
        out_specs=pl.BlockSpec((1, TL, D), lambda b, i: (b, i, 0)),
        out_shape=jax.ShapeDtypeStruct((B, L, D), jnp.float32),
    )(q_all[:, :L], kg, vg, kp[:, :L], vp[:, :L], gk, gv,
      distances, params, Wout, bout8)
    return out
